# baseline (speedup 1.0000x reference)
import functools

import jax
import jax.numpy as jnp
import numpy as np
from jax.experimental import pallas as pl
from jax.experimental.pallas import tpu as pltpu

N = 10000
E = 160000
A = 160000
D_IN = 128
D_OUT = 32
H = 4
HO = 128

BN = 1000
BE = 2000
NEG_BIG = -1e30


def _silu(x):
    return x * (1.0 / (1.0 + jnp.exp(-x)))


def _k1_body(x_ref, w_ref, b_ref, o_ref):
    o_ref[...] = jnp.dot(x_ref[...], w_ref[...],
                         preferred_element_type=jnp.float32) + b_ref[...]


def _node_precompute(x, wcat, bcat):
    grid = (N // BN,)
    return pl.pallas_call(
        _k1_body,
        grid=grid,
        in_specs=[
            pl.BlockSpec((BN, D_IN), lambda i: (i, 0)),
            pl.BlockSpec((D_IN, wcat.shape[1]), lambda i: (0, 0)),
            pl.BlockSpec((1, wcat.shape[1]), lambda i: (0, 0)),
        ],
        out_specs=pl.BlockSpec((BN, wcat.shape[1]), lambda i: (i, 0)),
        out_shape=jax.ShapeDtypeStruct((N, wcat.shape[1]), jnp.float32),
    )(x, wcat, bcat)


def _k3_body(g_ref, pd_ref, ea_ref, wc_ref, wrad_ref, w2_ref, b2_ref,
             avec_ref, s_ref, f_ref, e_ref, dn_ref, m_ref):
    i = pl.program_id(0)

    @pl.when(i == 0)
    def _():
        m_ref[...] = jnp.full_like(m_ref, NEG_BIG)

    pd = pd_ref[...]
    radial2 = jnp.sum(pd * pd, axis=1, keepdims=True)
    radial = jnp.sqrt(radial2)
    dn_ref[...] = pd * (1.0 / (jnp.sqrt(radial) + 1e-16))

    f_pre = (g_ref[...]
             + jnp.dot(ea_ref[...], wc_ref[...], preferred_element_type=jnp.float32)
             + radial * wrad_ref[...])
    f = jnp.dot(_silu(f_pre), w2_ref[...], preferred_element_type=jnp.float32) + b2_ref[...]
    f_ref[...] = f
    lf = jnp.where(f >= 0, f, 0.2 * f)
    e = jnp.dot(lf * avec_ref[...], s_ref[...], preferred_element_type=jnp.float32)
    e_ref[...] = e
    m_ref[...] = jnp.maximum(m_ref[...], jnp.max(e))


def _edge_pass1(g, pd, ea, wc, wrad, w2, b2, avec, sel, nE, bE):
    grid = (nE // bE,)
    dea = ea.shape[1]
    return pl.pallas_call(
        _k3_body,
        grid=grid,
        in_specs=[
            pl.BlockSpec((bE, HO), lambda i: (i, 0)),
            pl.BlockSpec((bE, 4), lambda i: (i, 0)),
            pl.BlockSpec((bE, dea), lambda i: (i, 0)),
            pl.BlockSpec((dea, HO), lambda i: (0, 0)),
            pl.BlockSpec((1, HO), lambda i: (0, 0)),
            pl.BlockSpec((HO, HO), lambda i: (0, 0)),
            pl.BlockSpec((1, HO), lambda i: (0, 0)),
            pl.BlockSpec((1, HO), lambda i: (0, 0)),
            pl.BlockSpec((HO, H), lambda i: (0, 0)),
        ],
        out_specs=[
            pl.BlockSpec((bE, HO), lambda i: (i, 0)),
            pl.BlockSpec((bE, H), lambda i: (i, 0)),
            pl.BlockSpec((bE, 4), lambda i: (i, 0)),
            pl.BlockSpec((1, 16), lambda i: (0, 0)),
        ],
        out_shape=[
            jax.ShapeDtypeStruct((nE, HO), jnp.float32),
            jax.ShapeDtypeStruct((nE, H), jnp.float32),
            jax.ShapeDtypeStruct((nE, 4), jnp.float32),
            jax.ShapeDtypeStruct((1, 16), jnp.float32),
        ],
    )(g, pd, ea, wc, wrad, w2, b2, avec, sel)


def _k6_body(f_ref, a_ref, dn_ref, st_ref, wp1_ref, bp1_ref, wp2_ref,
             proj_ref, tr_ref):
    a_spread = jnp.dot(a_ref[...], st_ref[...], preferred_element_type=jnp.float32)
    proj = f_ref[...] * a_spread
    proj_ref[...] = proj
    s = jnp.dot(_silu(jnp.dot(proj, wp1_ref[...], preferred_element_type=jnp.float32)
                      + bp1_ref[...]),
                wp2_ref[...], preferred_element_type=jnp.float32)
    lane = jax.lax.broadcasted_iota(jnp.int32, tr_ref.shape, 1)
    tr_ref[...] = jnp.where(lane == 3, 1.0, dn_ref[...] * s)


def _edge_pass2(f, a4, dn, selT, wp1, bp1, wp2, nE, bE):
    grid = (nE // bE,)
    return pl.pallas_call(
        _k6_body,
        grid=grid,
        in_specs=[
            pl.BlockSpec((bE, HO), lambda i: (i, 0)),
            pl.BlockSpec((bE, H), lambda i: (i, 0)),
            pl.BlockSpec((bE, 4), lambda i: (i, 0)),
            pl.BlockSpec((H, HO), lambda i: (0, 0)),
            pl.BlockSpec((HO, HO), lambda i: (0, 0)),
            pl.BlockSpec((1, HO), lambda i: (0, 0)),
            pl.BlockSpec((HO, 1), lambda i: (0, 0)),
        ],
        out_specs=[
            pl.BlockSpec((bE, HO), lambda i: (i, 0)),
            pl.BlockSpec((bE, 4), lambda i: (i, 0)),
        ],
        out_shape=[
            jax.ShapeDtypeStruct((nE, HO), jnp.float32),
            jax.ShapeDtypeStruct((nE, 4), jnp.float32),
        ],
    )(f, a4, dn, selT, wp1, bp1, wp2)


def _k8_body(x_ref, pos_ref, agge_ref, agga_ref, ce_ref, ca_ref,
             w1x_ref, w1a_ref, b1_ref, w2_ref, b2_ref,
             wse_ref, wsa_ref, wsx_ref, bsum_ref,
             gm_ref, gsp_ref, bsp_ref, prelu_ref,
             wpos_ref, bpos_ref,
             ox_ref, op_ref):
    x = x_ref[...]
    xw = jnp.dot(x, w1x_ref[...], preferred_element_type=jnp.float32) + b1_ref[...]

    def upd(agg):
        z = _silu(xw + jnp.dot(agg, w1a_ref[...], preferred_element_type=jnp.float32))
        return jnp.dot(z, w2_ref[...], preferred_element_type=jnp.float32) + b2_ref[...]

    ne = upd(agge_ref[...])
    na = upd(agga_ref[...])
    h = (jnp.dot(ne, wse_ref[...], preferred_element_type=jnp.float32)
         + jnp.dot(na, wsa_ref[...], preferred_element_type=jnp.float32)
         + jnp.dot(x, wsx_ref[...], preferred_element_type=jnp.float32)
         + bsum_ref[...] + x)
    mu = jnp.dot(h, gm_ref[...], preferred_element_type=jnp.float32)
    d = h - mu
    var = jnp.dot(d * d, gm_ref[...], preferred_element_type=jnp.float32)
    hn = d * jax.lax.rsqrt(var + 1e-5) * gsp_ref[...] + bsp_ref[...]
    ox_ref[...] = jnp.where(hn >= 0, hn, prelu_ref[0, 0] * hn)

    pos = pos_ref[...]
    ce = ce_ref[...]
    ca = ca_ref[...]
    cnt_e = jnp.maximum(ce[:, 3:4], 1.0)
    cnt_a = jnp.maximum(ca[:, 3:4], 1.0)
    pe = pos + ce / cnt_e
    pa = pos + ca / cnt_a
    pcat = jnp.concatenate([pe, pa, pos], axis=1)
    op_ref[...] = jnp.dot(pcat, wpos_ref[...], preferred_element_type=jnp.float32) + bpos_ref[...]


def _final(x, pos4, agg_e, agg_a, coord_e, coord_a,
           w1x, w1a, b1, w2, b2, wse, wsa, wsx, bsum,
           gm, gsp, bsp, prelu, wpos, bpos):
    grid = (N // BN,)
    full = lambda r, c: pl.BlockSpec((r, c), lambda i: (0, 0))
    return pl.pallas_call(
        _k8_body,
        grid=grid,
        in_specs=[
            pl.BlockSpec((BN, D_IN), lambda i: (i, 0)),
            pl.BlockSpec((BN, 4), lambda i: (i, 0)),
            pl.BlockSpec((BN, HO), lambda i: (i, 0)),
            pl.BlockSpec((BN, HO), lambda i: (i, 0)),
            pl.BlockSpec((BN, 4), lambda i: (i, 0)),
            pl.BlockSpec((BN, 4), lambda i: (i, 0)),
            full(D_IN, HO), full(HO, HO), full(1, HO), full(HO, HO), full(1, HO),
            full(HO, HO), full(HO, HO), full(D_IN, HO), full(1, HO),
            full(HO, HO), full(1, HO), full(1, HO),
            pl.BlockSpec(memory_space=pltpu.SMEM),
            full(12, 4), full(1, 4),
        ],
        out_specs=[
            pl.BlockSpec((BN, HO), lambda i: (i, 0)),
            pl.BlockSpec((BN, 4), lambda i: (i, 0)),
        ],
        out_shape=[
            jax.ShapeDtypeStruct((N, HO), jnp.float32),
            jax.ShapeDtypeStruct((N, 4), jnp.float32),
        ],
    )(x, pos4, agg_e, agg_a, coord_e, coord_a,
      w1x, w1a, b1, w2, b2, wse, wsa, wsx, bsum,
      gm, gsp, bsp, prelu, wpos, bpos)


def kernel(x, pos, edge_attr, angle_attr, angle_edge_attr, params, edge_index, angle_index):
    p = params
    We1, be1 = p["edge1"]["w"], p["edge1"]["b"]
    Wa1, ba1 = p["ang1"]["w"], p["ang1"]["b"]

    wcat = jnp.concatenate([We1[0:128], We1[128:256], Wa1[0:128], Wa1[128:256], Wa1[256:384]], axis=1)
    bcat = jnp.concatenate([jnp.zeros_like(be1), be1, jnp.zeros_like(ba1), jnp.zeros_like(ba1), ba1])[None]
    xn = _node_precompute(x, wcat, bcat)

    pos4 = jnp.pad(pos, ((0, 0), (0, 1)))

    sel = jnp.asarray(np.kron(np.eye(H, dtype=np.float32), np.ones((D_OUT, 1), np.float32)))
    selT = sel.T

    i_, j_ = edge_index[0], edge_index[1]
    g_e = xn[i_, 0:128] + xn[j_, 128:256]
    pd_e = pos4[j_] - pos4[i_]
    avec_e = p["a_edge"].reshape(1, HO)
    f_e, e_e, dn_e, m_e = _edge_pass1(
        g_e, pd_e, edge_attr, We1[256:272], We1[272:273],
        p["edge2"]["w"], p["edge2"]["b"][None], avec_e, sel, E, BE)
    M_e = m_e[0, 0]
    exp_e = jnp.exp(e_e - M_e)
    den_e = jax.ops.segment_sum(exp_e, j_, num_segments=N)
    a_e = exp_e / (den_e[j_] + 1e-16)
    proj_e, tr_e = _edge_pass2(f_e, a_e, dn_e, selT,
                               p["upd_pos1"]["w"], p["upd_pos1"]["b"][None],
                               p["upd_pos2"]["w"], E, BE)
    agg_e = jax.ops.segment_sum(proj_e, j_, num_segments=N)
    coord_e = jax.ops.segment_sum(tr_e, j_, num_segments=N)

    ai, ah, aj = angle_index[0], angle_index[1], angle_index[2]
    g_a = xn[ai, 256:384] + xn[ah, 384:512] + xn[aj, 512:640]
    pd_a = pos4[aj] - pos4[ai]
    ea_a = jnp.concatenate([angle_edge_attr[:, 0], angle_edge_attr[:, 1],
                            angle_attr * (np.pi / 180.0)], axis=1)
    wc_a = jnp.concatenate([Wa1[384:400], Wa1[400:416], Wa1[416:417]], axis=0)
    avec_a = p["a_angle"].reshape(1, HO)
    f_a, e_a, dn_a, m_a = _edge_pass1(
        g_a, pd_a, ea_a, wc_a, Wa1[417:418],
        p["ang2"]["w"], p["ang2"]["b"][None], avec_a, sel, A, BE)
    M_a = m_a[0, 0]
    exp_a = jnp.exp(e_a - M_a)
    den_a = jax.ops.segment_sum(exp_a, aj, num_segments=N)
    a_a = exp_a / (den_a[aj] + 1e-16)
    proj_a, tr_a = _edge_pass2(f_a, a_a, dn_a, selT,
                               p["upd_pos1"]["w"], p["upd_pos1"]["b"][None],
                               p["upd_pos2"]["w"], A, BE)
    agg_a = jax.ops.segment_sum(proj_a, aj, num_segments=N)
    coord_a = jax.ops.segment_sum(tr_a, aj, num_segments=N)

    Wu1 = p["upd_node1"]["w"]
    bsum = (p["sum_edge"]["b"] + p["sum_angle"]["b"] + p["sum_x"]["b"])[None]
    gm = jnp.asarray(np.kron(np.eye(H, dtype=np.float32), np.full((D_OUT, D_OUT), 1.0 / D_OUT, np.float32)))
    gsp = jnp.tile(p["ln_g"], H)[None] * 1.0
    bsp = jnp.tile(p["ln_b"], H)[None]
    bias_sp = jnp.tile(p["bias"], H)[None]
    bsum = bsum + bias_sp
    prelu = p["prelu"].reshape(1, 1)
    wpos = jnp.zeros((12, 4), jnp.float32)
    wpos = wpos.at[0:3, 0:3].set(p["sum_edge_pos"]["w"])
    wpos = wpos.at[4:7, 0:3].set(p["sum_angle_pos"]["w"])
    wpos = wpos.at[8:11, 0:3].set(p["sum_x_pos"]["w"])
    bpos = jnp.pad(p["sum_edge_pos"]["b"] + p["sum_angle_pos"]["b"] + p["sum_x_pos"]["b"], (0, 1))[None]

    out_x, out_pos4 = _final(
        x, pos4, agg_e, agg_a, coord_e, coord_a,
        Wu1[0:128], Wu1[128:256], p["upd_node1"]["b"][None],
        p["upd_node2"]["w"], p["upd_node2"]["b"][None],
        p["sum_edge"]["w"], p["sum_angle"]["w"], p["sum_x"]["w"], bsum,
        gm, gsp, bsp, prelu, wpos, bpos)
    return out_x, out_pos4[:, 0:3]


# speedup vs baseline: 89.3218x; 89.3218x over previous
import functools

import jax
import jax.numpy as jnp
import numpy as np
from jax.experimental import pallas as pl
from jax.experimental.pallas import tpu as pltpu

N = 10000
E = 160000
A = 160000
D_IN = 128
D_OUT = 32
H = 4
HO = 128

BN = 1000
BE = 2000
NEG_BIG = -1e30


def _silu(x):
    return x * (1.0 / (1.0 + jnp.exp(-x)))


def _k1_body(x_ref, w_ref, b_ref, o_ref):
    o_ref[...] = jnp.dot(x_ref[...], w_ref[...],
                         preferred_element_type=jnp.float32) + b_ref[...]


def _node_precompute(x, wcat, bcat):
    grid = (N // BN,)
    return pl.pallas_call(
        _k1_body,
        grid=grid,
        in_specs=[
            pl.BlockSpec((BN, D_IN), lambda i: (i, 0)),
            pl.BlockSpec((D_IN, wcat.shape[1]), lambda i: (0, 0)),
            pl.BlockSpec((1, wcat.shape[1]), lambda i: (0, 0)),
        ],
        out_specs=pl.BlockSpec((BN, wcat.shape[1]), lambda i: (i, 0)),
        out_shape=jax.ShapeDtypeStruct((N, wcat.shape[1]), jnp.float32),
    )(x, wcat, bcat)


def _k3_body(g_ref, pd_ref, ea_ref, wc_ref, wrad_ref, w2_ref, b2_ref,
             avec_ref, s_ref, f_ref, e_ref, dn_ref, m_ref):
    i = pl.program_id(0)

    @pl.when(i == 0)
    def _():
        m_ref[...] = jnp.full_like(m_ref, NEG_BIG)

    pd = pd_ref[...]
    radial2 = jnp.sum(pd * pd, axis=1, keepdims=True)
    radial = jnp.sqrt(radial2)
    dn_ref[...] = pd * (1.0 / (jnp.sqrt(radial) + 1e-16))

    f_pre = (g_ref[...]
             + jnp.dot(ea_ref[...], wc_ref[...], preferred_element_type=jnp.float32)
             + radial * wrad_ref[...])
    f = jnp.dot(_silu(f_pre), w2_ref[...], preferred_element_type=jnp.float32) + b2_ref[...]
    f_ref[...] = f
    lf = jnp.where(f >= 0, f, 0.2 * f)
    e = jnp.dot(lf * avec_ref[...], s_ref[...], preferred_element_type=jnp.float32)
    e_ref[...] = e
    m_ref[...] = jnp.maximum(m_ref[...], jnp.max(e))


def _edge_pass1(g, pd, ea, wc, wrad, w2, b2, avec, sel, nE, bE):
    grid = (nE // bE,)
    dea = ea.shape[1]
    return pl.pallas_call(
        _k3_body,
        grid=grid,
        in_specs=[
            pl.BlockSpec((bE, HO), lambda i: (i, 0)),
            pl.BlockSpec((bE, 4), lambda i: (i, 0)),
            pl.BlockSpec((bE, dea), lambda i: (i, 0)),
            pl.BlockSpec((dea, HO), lambda i: (0, 0)),
            pl.BlockSpec((1, HO), lambda i: (0, 0)),
            pl.BlockSpec((HO, HO), lambda i: (0, 0)),
            pl.BlockSpec((1, HO), lambda i: (0, 0)),
            pl.BlockSpec((1, HO), lambda i: (0, 0)),
            pl.BlockSpec((HO, H), lambda i: (0, 0)),
        ],
        out_specs=[
            pl.BlockSpec((bE, HO), lambda i: (i, 0)),
            pl.BlockSpec((bE, H), lambda i: (i, 0)),
            pl.BlockSpec((bE, 4), lambda i: (i, 0)),
            pl.BlockSpec((1, 16), lambda i: (0, 0)),
        ],
        out_shape=[
            jax.ShapeDtypeStruct((nE, HO), jnp.float32),
            jax.ShapeDtypeStruct((nE, H), jnp.float32),
            jax.ShapeDtypeStruct((nE, 4), jnp.float32),
            jax.ShapeDtypeStruct((1, 16), jnp.float32),
        ],
    )(g, pd, ea, wc, wrad, w2, b2, avec, sel)


def _k6_body(f_ref, a_ref, dn_ref, st_ref, wp1_ref, bp1_ref, wp2_ref,
             proj_ref, tr_ref):
    a_spread = jnp.dot(a_ref[...], st_ref[...], preferred_element_type=jnp.float32)
    proj = f_ref[...] * a_spread
    proj_ref[...] = proj
    s = jnp.dot(_silu(jnp.dot(proj, wp1_ref[...], preferred_element_type=jnp.float32)
                      + bp1_ref[...]),
                wp2_ref[...], preferred_element_type=jnp.float32)
    lane = jax.lax.broadcasted_iota(jnp.int32, tr_ref.shape, 1)
    tr_ref[...] = jnp.where(lane == 3, 1.0, dn_ref[...] * s)


def _edge_pass2(f, a4, dn, selT, wp1, bp1, wp2, nE, bE):
    grid = (nE // bE,)
    return pl.pallas_call(
        _k6_body,
        grid=grid,
        in_specs=[
            pl.BlockSpec((bE, HO), lambda i: (i, 0)),
            pl.BlockSpec((bE, H), lambda i: (i, 0)),
            pl.BlockSpec((bE, 4), lambda i: (i, 0)),
            pl.BlockSpec((H, HO), lambda i: (0, 0)),
            pl.BlockSpec((HO, HO), lambda i: (0, 0)),
            pl.BlockSpec((1, HO), lambda i: (0, 0)),
            pl.BlockSpec((HO, 1), lambda i: (0, 0)),
        ],
        out_specs=[
            pl.BlockSpec((bE, HO), lambda i: (i, 0)),
            pl.BlockSpec((bE, 4), lambda i: (i, 0)),
        ],
        out_shape=[
            jax.ShapeDtypeStruct((nE, HO), jnp.float32),
            jax.ShapeDtypeStruct((nE, 4), jnp.float32),
        ],
    )(f, a4, dn, selT, wp1, bp1, wp2)


def _k8_body(x_ref, pos_ref, agge_ref, agga_ref, ce_ref, ca_ref,
             w1x_ref, w1a_ref, b1_ref, w2_ref, b2_ref,
             wse_ref, wsa_ref, wsx_ref, bsum_ref,
             gm_ref, gsp_ref, bsp_ref, prelu_ref,
             wpos_ref, bpos_ref,
             ox_ref, op_ref):
    x = x_ref[...]
    xw = jnp.dot(x, w1x_ref[...], preferred_element_type=jnp.float32) + b1_ref[...]

    def upd(agg):
        z = _silu(xw + jnp.dot(agg, w1a_ref[...], preferred_element_type=jnp.float32))
        return jnp.dot(z, w2_ref[...], preferred_element_type=jnp.float32) + b2_ref[...]

    ne = upd(agge_ref[...])
    na = upd(agga_ref[...])
    h = (jnp.dot(ne, wse_ref[...], preferred_element_type=jnp.float32)
         + jnp.dot(na, wsa_ref[...], preferred_element_type=jnp.float32)
         + jnp.dot(x, wsx_ref[...], preferred_element_type=jnp.float32)
         + bsum_ref[...] + x)
    mu = jnp.dot(h, gm_ref[...], preferred_element_type=jnp.float32)
    d = h - mu
    var = jnp.dot(d * d, gm_ref[...], preferred_element_type=jnp.float32)
    hn = d * jax.lax.rsqrt(var + 1e-5) * gsp_ref[...] + bsp_ref[...]
    ox_ref[...] = jnp.where(hn >= 0, hn, prelu_ref[0, 0] * hn)

    pos = pos_ref[...]
    ce = ce_ref[...]
    ca = ca_ref[...]
    cnt_e = jnp.maximum(ce[:, 3:4], 1.0)
    cnt_a = jnp.maximum(ca[:, 3:4], 1.0)
    pe = pos + ce / cnt_e
    pa = pos + ca / cnt_a
    pcat = jnp.concatenate([pe, pa, pos], axis=1)
    op_ref[...] = jnp.dot(pcat, wpos_ref[...], preferred_element_type=jnp.float32) + bpos_ref[...]


def _final(x, pos4, agg_e, agg_a, coord_e, coord_a,
           w1x, w1a, b1, w2, b2, wse, wsa, wsx, bsum,
           gm, gsp, bsp, prelu, wpos, bpos):
    grid = (N // BN,)
    full = lambda r, c: pl.BlockSpec((r, c), lambda i: (0, 0))
    return pl.pallas_call(
        _k8_body,
        grid=grid,
        in_specs=[
            pl.BlockSpec((BN, D_IN), lambda i: (i, 0)),
            pl.BlockSpec((BN, 4), lambda i: (i, 0)),
            pl.BlockSpec((BN, HO), lambda i: (i, 0)),
            pl.BlockSpec((BN, HO), lambda i: (i, 0)),
            pl.BlockSpec((BN, 4), lambda i: (i, 0)),
            pl.BlockSpec((BN, 4), lambda i: (i, 0)),
            full(D_IN, HO), full(HO, HO), full(1, HO), full(HO, HO), full(1, HO),
            full(HO, HO), full(HO, HO), full(D_IN, HO), full(1, HO),
            full(HO, HO), full(1, HO), full(1, HO),
            pl.BlockSpec(memory_space=pltpu.SMEM),
            full(12, 4), full(1, 4),
        ],
        out_specs=[
            pl.BlockSpec((BN, HO), lambda i: (i, 0)),
            pl.BlockSpec((BN, 4), lambda i: (i, 0)),
        ],
        out_shape=[
            jax.ShapeDtypeStruct((N, HO), jnp.float32),
            jax.ShapeDtypeStruct((N, 4), jnp.float32),
        ],
    )(x, pos4, agg_e, agg_a, coord_e, coord_a,
      w1x, w1a, b1, w2, b2, wse, wsa, wsx, bsum,
      gm, gsp, bsp, prelu, wpos, bpos)


def kernel(x, pos, edge_attr, angle_attr, angle_edge_attr, params, edge_index, angle_index):
    p = params
    We1, be1 = p["edge1"]["w"], p["edge1"]["b"]
    Wa1, ba1 = p["ang1"]["w"], p["ang1"]["b"]

    wcat = jnp.concatenate([We1[0:128], We1[128:256], Wa1[0:128], Wa1[128:256], Wa1[256:384]], axis=1)
    bcat = jnp.concatenate([jnp.zeros_like(be1), be1, jnp.zeros_like(ba1), jnp.zeros_like(ba1), ba1])[None]
    xn = _node_precompute(x, wcat, bcat)
    xe_i_t = jnp.copy(xn[:, 0:128])
    xe_j_t = jnp.copy(xn[:, 128:256])
    xa_i_t = jnp.copy(xn[:, 256:384])
    xa_h_t = jnp.copy(xn[:, 384:512])
    xa_j_t = jnp.copy(xn[:, 512:640])

    pos4 = jnp.pad(pos, ((0, 0), (0, 1)))

    sel = jnp.asarray(np.kron(np.eye(H, dtype=np.float32), np.ones((D_OUT, 1), np.float32)))
    selT = sel.T

    i_, j_ = edge_index[0], edge_index[1]
    g_e = xe_i_t[i_] + xe_j_t[j_]
    pd_e = pos4[j_] - pos4[i_]
    avec_e = p["a_edge"].reshape(1, HO)
    f_e, e_e, dn_e, m_e = _edge_pass1(
        g_e, pd_e, edge_attr, We1[256:272], We1[272:273],
        p["edge2"]["w"], p["edge2"]["b"][None], avec_e, sel, E, BE)
    M_e = m_e[0, 0]
    exp_e = jnp.exp(e_e - M_e)
    den_e = jax.ops.segment_sum(exp_e, j_, num_segments=N)
    a_e = exp_e / (den_e[j_] + 1e-16)
    proj_e, tr_e = _edge_pass2(f_e, a_e, dn_e, selT,
                               p["upd_pos1"]["w"], p["upd_pos1"]["b"][None],
                               p["upd_pos2"]["w"], E, BE)
    agg_e = jax.ops.segment_sum(proj_e, j_, num_segments=N)
    coord_e = jax.ops.segment_sum(tr_e, j_, num_segments=N)

    ai, ah, aj = angle_index[0], angle_index[1], angle_index[2]
    g_a = xa_i_t[ai] + xa_h_t[ah] + xa_j_t[aj]
    pd_a = pos4[aj] - pos4[ai]
    ea_a = jnp.concatenate([angle_edge_attr[:, 0], angle_edge_attr[:, 1],
                            angle_attr * (np.pi / 180.0)], axis=1)
    wc_a = jnp.concatenate([Wa1[384:400], Wa1[400:416], Wa1[416:417]], axis=0)
    avec_a = p["a_angle"].reshape(1, HO)
    f_a, e_a, dn_a, m_a = _edge_pass1(
        g_a, pd_a, ea_a, wc_a, Wa1[417:418],
        p["ang2"]["w"], p["ang2"]["b"][None], avec_a, sel, A, BE)
    M_a = m_a[0, 0]
    exp_a = jnp.exp(e_a - M_a)
    den_a = jax.ops.segment_sum(exp_a, aj, num_segments=N)
    a_a = exp_a / (den_a[aj] + 1e-16)
    proj_a, tr_a = _edge_pass2(f_a, a_a, dn_a, selT,
                               p["upd_pos1"]["w"], p["upd_pos1"]["b"][None],
                               p["upd_pos2"]["w"], A, BE)
    agg_a = jax.ops.segment_sum(proj_a, aj, num_segments=N)
    coord_a = jax.ops.segment_sum(tr_a, aj, num_segments=N)

    Wu1 = p["upd_node1"]["w"]
    bsum = (p["sum_edge"]["b"] + p["sum_angle"]["b"] + p["sum_x"]["b"])[None]
    gm = jnp.asarray(np.kron(np.eye(H, dtype=np.float32), np.full((D_OUT, D_OUT), 1.0 / D_OUT, np.float32)))
    gsp = jnp.tile(p["ln_g"], H)[None] * 1.0
    bsp = jnp.tile(p["ln_b"], H)[None]
    bias_sp = jnp.tile(p["bias"], H)[None]
    bsum = bsum + bias_sp
    prelu = p["prelu"].reshape(1, 1)
    wpos = jnp.zeros((12, 4), jnp.float32)
    wpos = wpos.at[0:3, 0:3].set(p["sum_edge_pos"]["w"])
    wpos = wpos.at[4:7, 0:3].set(p["sum_angle_pos"]["w"])
    wpos = wpos.at[8:11, 0:3].set(p["sum_x_pos"]["w"])
    bpos = jnp.pad(p["sum_edge_pos"]["b"] + p["sum_angle_pos"]["b"] + p["sum_x_pos"]["b"], (0, 1))[None]

    out_x, out_pos4 = _final(
        x, pos4, agg_e, agg_a, coord_e, coord_a,
        Wu1[0:128], Wu1[128:256], p["upd_node1"]["b"][None],
        p["upd_node2"]["w"], p["upd_node2"]["b"][None],
        p["sum_edge"]["w"], p["sum_angle"]["w"], p["sum_x"]["w"], bsum,
        gm, gsp, bsp, prelu, wpos, bpos)
    return out_x, out_pos4[:, 0:3]


# speedup vs baseline: 399.1062x; 4.4682x over previous
import functools

import jax
import jax.numpy as jnp
import numpy as np
from jax import lax
from jax.experimental import pallas as pl
from jax.experimental.pallas import tpu as pltpu
from jax.experimental.pallas import tpu_sc as plsc

N = 10000
E = 160000
A = 160000
D_IN = 128
D_OUT = 32
H = 4
HO = 128

BN = 1000
BE = 3200
CH = 128
NW = 32
NEG_BIG = -1e30
_SC_PARAMS = None


def _silu(x):
    return x * (1.0 / (1.0 + jnp.exp(-x)))


def _mesh():
    return plsc.VectorSubcoreMesh(core_axis_name="c", subcore_axis_name="s")


def _wid():
    return lax.axis_index("s") * 2 + lax.axis_index("c")


def _chunk_count(nch):
    nfull, extra = nch // NW, nch % NW
    return nfull + jnp.where(_wid() < extra, 1, 0)


def _sc_cp():
    return pltpu.CompilerParams(needs_layout_passes=False)


def _make_gather(n_tabs, nE):
    nch = nE // CH

    def body(*refs):
        tabs = refs[:n_tabs]
        (idx, px, py, pz, gx, pdT,
         gbuf, pv0, pv1, pv2, pdb0, pdb1, pdb2, *ivs) = refs[n_tabs:]
        pvs = (pv0, pv1, pv2)
        pdbs = (pdb0, pdb1, pdb2)
        wid = _wid()
        pltpu.sync_copy(px, pv0)
        pltpu.sync_copy(py, pv1)
        pltpu.sync_copy(pz, pv2)

        def step(k, _):
            base = pl.multiple_of((wid + k * NW) * CH, CH)
            for t in range(n_tabs):
                pltpu.sync_copy(idx.at[pl.ds(t, 1), pl.ds(base, CH)], ivs[t])
            pltpu.sync_copy(tabs[0].at[ivs[0].at[0]], gbuf)
            for t in range(1, n_tabs):
                pltpu.sync_copy(tabs[t].at[ivs[t].at[0]], gbuf, add=True)
            pltpu.sync_copy(gbuf, gx.at[pl.ds(base, CH)])
            for l in range(CH // 16):
                sl = pl.ds(l * 16, 16)
                iv16 = ivs[0][0, sl]
                jv16 = ivs[n_tabs - 1][0, sl]
                for cc in range(3):
                    pdbs[cc][0, sl] = (plsc.load_gather(pvs[cc], [jv16])
                                       - plsc.load_gather(pvs[cc], [iv16]))
            for cc in range(3):
                pltpu.sync_copy(pdbs[cc], pdT.at[pl.ds(cc, 1), pl.ds(base, CH)])
            return 0

        lax.fori_loop(0, _chunk_count(nch), step, 0)

    return functools.partial(
        pl.kernel, body, mesh=_mesh(), compiler_params=_sc_cp(),
        out_type=[
            jax.ShapeDtypeStruct((nE, HO), jnp.float32),
            jax.ShapeDtypeStruct((3, nE), jnp.float32),
        ],
        scratch_types=(
            [pltpu.VMEM((CH, HO), jnp.float32)]
            + [pltpu.VMEM((N,), jnp.float32)] * 3
            + [pltpu.VMEM((1, CH), jnp.float32)] * 3
            + [pltpu.VMEM((1, CH), jnp.int32)] * n_tabs
        ),
    )()


def _make_denom(nE):
    nch = nE // CH

    def body(eT, jidx, m, zn, d0, d1, d2, d3,
             jv, eb0, eb1, eb2, eb3, mbuf, den0, den1, den2, den3):
        c = lax.axis_index("c")
        s = lax.axis_index("s")
        wid = _wid()
        ebs = (eb0, eb1, eb2, eb3)
        dens = (den0, den1, den2, den3)
        outs = (d0, d1, d2, d3)

        @pl.when(s == 0)
        def _():
            for t in range(H):
                pltpu.sync_copy(zn, dens[t])

        pltpu.sync_copy(m, mbuf)
        plsc.subcore_barrier()
        mvec = mbuf[0]

        def step(k, _):
            base = pl.multiple_of((wid + k * NW) * CH, CH)
            pltpu.sync_copy(jidx.at[pl.ds(base, CH)], jv.at[0])
            for cc in range(H):
                pltpu.sync_copy(eT.at[pl.ds(cc, 1), pl.ds(base, CH)], ebs[cc])
                for l in range(CH // 16):
                    sl = pl.ds(l * 16, 16)
                    ebs[cc][0, sl] = jnp.exp(ebs[cc][0, sl] - mvec)
                pltpu.sync_copy(ebs[cc].at[0], dens[cc].at[jv.at[0]], add=True)
            return 0

        lax.fori_loop(0, _chunk_count(nch), step, 0)
        plsc.subcore_barrier()

        @pl.when(s == 0)
        def _():
            for t in range(H):
                pltpu.sync_copy(dens[t], outs[t].at[c])

    return functools.partial(
        pl.kernel, body, mesh=_mesh(), compiler_params=_sc_cp(),
        out_type=[jax.ShapeDtypeStruct((2, N), jnp.float32)] * H,
        scratch_types=(
            [pltpu.VMEM((1, CH), jnp.int32)]
            + [pltpu.VMEM((1, CH), jnp.float32)] * H
            + [pltpu.VMEM((1, 16), jnp.float32)]
            + [pltpu.VMEM_SHARED((N,), jnp.float32)] * H
        ),
    )()


def _make_dgather(nE):
    nch = nE // CH

    def body(d00, d01, d02, d03, d10, d11, d12, d13, jidx, out,
             jv, db0, db1, db2, db3, v00, v01, v02, v03, v10, v11, v12, v13):
        wid = _wid()
        dbs = (db0, db1, db2, db3)
        srcs = (d00, d01, d02, d03, d10, d11, d12, d13)
        vs = (v00, v01, v02, v03, v10, v11, v12, v13)
        for t in range(8):
            pltpu.sync_copy(srcs[t], vs[t])

        def step(k, _):
            base = pl.multiple_of((wid + k * NW) * CH, CH)
            pltpu.sync_copy(jidx.at[pl.ds(base, CH)], jv.at[0])
            for l in range(CH // 16):
                sl = pl.ds(l * 16, 16)
                jv16 = jv[0, sl]
                for cc in range(H):
                    dbs[cc][0, sl] = (plsc.load_gather(vs[cc], [jv16])
                                      + plsc.load_gather(vs[cc + 4], [jv16]))
            for cc in range(H):
                pltpu.sync_copy(dbs[cc], out.at[pl.ds(cc, 1), pl.ds(base, CH)])
            return 0

        lax.fori_loop(0, _chunk_count(nch), step, 0)

    return functools.partial(
        pl.kernel, body, mesh=_mesh(), compiler_params=_sc_cp(),
        out_type=jax.ShapeDtypeStruct((H, nE), jnp.float32),
        scratch_types=(
            [pltpu.VMEM((1, CH), jnp.int32)]
            + [pltpu.VMEM((1, CH), jnp.float32)] * H
            + [pltpu.VMEM((N,), jnp.float32)] * 8
        ),
    )()


def _make_scatter(nE):
    nch = nE // CH
    SLAB = 640

    def body(proj, trT, jidx, zN128, zn, aggp, c0, c1, c2, c3,
             jv, pbuf, tb0, tb1, tb2, tb3, agg, co0, co1, co2, co3):
        c = lax.axis_index("c")
        s = lax.axis_index("s")
        wid = _wid()
        tbs = (tb0, tb1, tb2, tb3)
        cos = (co0, co1, co2, co3)
        couts = (c0, c1, c2, c3)

        @pl.when(s == 0)
        def _():
            for t in range(4):
                pltpu.sync_copy(zn, cos[t])

        off = pl.multiple_of(s * SLAB, 8)

        @pl.when(s < 15)
        def _():
            pltpu.sync_copy(zN128.at[pl.ds(off, SLAB)], agg.at[pl.ds(off, SLAB)])

        @pl.when(s == 15)
        def _():
            pltpu.sync_copy(zN128.at[pl.ds(15 * SLAB, N - 15 * SLAB)],
                            agg.at[pl.ds(15 * SLAB, N - 15 * SLAB)])

        plsc.subcore_barrier()

        def step(k, _):
            base = pl.multiple_of((wid + k * NW) * CH, CH)
            pltpu.sync_copy(jidx.at[pl.ds(base, CH)], jv.at[0])
            pltpu.sync_copy(proj.at[pl.ds(base, CH)], pbuf)
            pltpu.sync_copy(pbuf, agg.at[jv.at[0]], add=True)
            for cc in range(4):
                pltpu.sync_copy(trT.at[pl.ds(cc, 1), pl.ds(base, CH)], tbs[cc])
                pltpu.sync_copy(tbs[cc].at[0], cos[cc].at[jv.at[0]], add=True)
            return 0

        lax.fori_loop(0, _chunk_count(nch), step, 0)
        plsc.subcore_barrier()
        offo = pl.multiple_of(c * N + s * SLAB, 8)

        @pl.when(s < 15)
        def _():
            pltpu.sync_copy(agg.at[pl.ds(off, SLAB)], aggp.at[pl.ds(offo, SLAB)])

        @pl.when(s == 15)
        def _():
            pltpu.sync_copy(agg.at[pl.ds(15 * SLAB, N - 15 * SLAB)],
                            aggp.at[pl.ds(pl.multiple_of(c * N + 15 * SLAB, 8), N - 15 * SLAB)])

        @pl.when(s == 0)
        def _():
            for t in range(4):
                pltpu.sync_copy(cos[t], couts[t].at[c])

    return functools.partial(
        pl.kernel, body, mesh=_mesh(), compiler_params=_sc_cp(),
        out_type=(
            [jax.ShapeDtypeStruct((2 * N, HO), jnp.float32)]
            + [jax.ShapeDtypeStruct((2, N), jnp.float32)] * 4
        ),
        scratch_types=(
            [pltpu.VMEM((1, CH), jnp.int32)]
            + [pltpu.VMEM((CH, HO), jnp.float32)]
            + [pltpu.VMEM((1, CH), jnp.float32)] * 4
            + [pltpu.VMEM_SHARED((N, HO), jnp.float32)]
            + [pltpu.VMEM_SHARED((N,), jnp.float32)] * 4
        ),
    )()


def _k1_body(x_ref, w_ref, b_ref, ei_ref, ej_ref, ai_ref, ah_ref, aj_ref):
    z = jnp.dot(x_ref[...], w_ref[...], preferred_element_type=jnp.float32) + b_ref[...]
    ei_ref[...] = z[:, 0:128]
    ej_ref[...] = z[:, 128:256]
    ai_ref[...] = z[:, 256:384]
    ah_ref[...] = z[:, 384:512]
    aj_ref[...] = z[:, 512:640]


def _node_tables(x, wcat, bcat):
    grid = (N // BN,)
    ospec = pl.BlockSpec((BN, HO), lambda i: (i, 0))
    oshape = jax.ShapeDtypeStruct((N, HO), jnp.float32)
    return pl.pallas_call(
        _k1_body,
        grid=grid,
        in_specs=[
            pl.BlockSpec((BN, D_IN), lambda i: (i, 0)),
            pl.BlockSpec((D_IN, 640), lambda i: (0, 0)),
            pl.BlockSpec((1, 640), lambda i: (0, 0)),
        ],
        out_specs=[ospec] * 5,
        out_shape=[oshape] * 5,
    )(x, wcat, bcat)


def _k3_body(gx_ref, pdT_ref, ea_ref, wc_ref, wrad_ref, w2_ref, b2_ref,
             avec_ref, s_ref, f_ref, eT_ref, dnT_ref, m_ref):
    i = pl.program_id(0)

    @pl.when(i == 0)
    def _():
        m_ref[...] = jnp.full_like(m_ref, NEG_BIG)

    pdT = pdT_ref[...]
    radial2 = (pdT[0:1] * pdT[0:1] + pdT[1:2] * pdT[1:2] + pdT[2:3] * pdT[2:3])
    radialT = jnp.sqrt(radial2)
    dnT_ref[...] = pdT * (1.0 / (jnp.sqrt(radialT) + 1e-16))

    rad_contrib = jax.lax.dot_general(radialT, wrad_ref[...],
                                      dimension_numbers=(((0,), (0,)), ((), ())),
                                      preferred_element_type=jnp.float32)
    f_pre = (gx_ref[...]
             + jnp.dot(ea_ref[...], wc_ref[...], preferred_element_type=jnp.float32)
             + rad_contrib)
    f = jnp.dot(_silu(f_pre), w2_ref[...], preferred_element_type=jnp.float32) + b2_ref[...]
    f_ref[...] = f
    lf = jnp.where(f >= 0, f, 0.2 * f)
    eT = jax.lax.dot_general(s_ref[...], lf * avec_ref[...],
                             dimension_numbers=(((0,), (1,)), ((), ())),
                             preferred_element_type=jnp.float32)
    eT_ref[...] = eT
    m_ref[...] = jnp.maximum(m_ref[...], jnp.max(eT))


def _edge_pass1(gx, pdT, ea, wc, wrad, w2, b2, avec, sel, nE):
    grid = (nE // BE,)
    dea = ea.shape[1]
    return pl.pallas_call(
        _k3_body,
        grid=grid,
        in_specs=[
            pl.BlockSpec((BE, HO), lambda i: (i, 0)),
            pl.BlockSpec((3, BE), lambda i: (0, i)),
            pl.BlockSpec((BE, dea), lambda i: (i, 0)),
            pl.BlockSpec((dea, HO), lambda i: (0, 0)),
            pl.BlockSpec((1, HO), lambda i: (0, 0)),
            pl.BlockSpec((HO, HO), lambda i: (0, 0)),
            pl.BlockSpec((1, HO), lambda i: (0, 0)),
            pl.BlockSpec((1, HO), lambda i: (0, 0)),
            pl.BlockSpec((HO, H), lambda i: (0, 0)),
        ],
        out_specs=[
            pl.BlockSpec((BE, HO), lambda i: (i, 0)),
            pl.BlockSpec((H, BE), lambda i: (0, i)),
            pl.BlockSpec((3, BE), lambda i: (0, i)),
            pl.BlockSpec((1, 16), lambda i: (0, 0)),
        ],
        out_shape=[
            jax.ShapeDtypeStruct((nE, HO), jnp.float32),
            jax.ShapeDtypeStruct((H, nE), jnp.float32),
            jax.ShapeDtypeStruct((3, nE), jnp.float32),
            jax.ShapeDtypeStruct((1, 16), jnp.float32),
        ],
    )(gx, pdT, ea, wc, wrad, w2, b2, avec, sel)


def _k6_body(f_ref, eT_ref, dT_ref, dnT_ref, m_ref, st_ref, wp1_ref, bp1_ref, wp2_ref,
             proj_ref, trT_ref):
    aT = jnp.exp(eT_ref[...] - m_ref[0, 0]) / (dT_ref[...] + 1e-16)
    a_spread = jax.lax.dot_general(aT, st_ref[...],
                                   dimension_numbers=(((0,), (0,)), ((), ())),
                                   preferred_element_type=jnp.float32)
    proj = f_ref[...] * a_spread
    proj_ref[...] = proj
    z = _silu(jnp.dot(proj, wp1_ref[...], preferred_element_type=jnp.float32)
              + bp1_ref[...])
    sT = jax.lax.dot_general(wp2_ref[...], z,
                             dimension_numbers=(((0,), (1,)), ((), ())),
                             preferred_element_type=jnp.float32)
    trT_ref[...] = jnp.concatenate(
        [dnT_ref[...] * sT, jnp.ones_like(sT)], axis=0)


def _edge_pass2(f, eT, dT, dnT, m, selT, wp1, bp1, wp2, nE):
    grid = (nE // BE,)
    return pl.pallas_call(
        _k6_body,
        grid=grid,
        in_specs=[
            pl.BlockSpec((BE, HO), lambda i: (i, 0)),
            pl.BlockSpec((H, BE), lambda i: (0, i)),
            pl.BlockSpec((H, BE), lambda i: (0, i)),
            pl.BlockSpec((3, BE), lambda i: (0, i)),
            pl.BlockSpec(memory_space=pltpu.SMEM),
            pl.BlockSpec((H, HO), lambda i: (0, 0)),
            pl.BlockSpec((HO, HO), lambda i: (0, 0)),
            pl.BlockSpec((1, HO), lambda i: (0, 0)),
            pl.BlockSpec((HO, 1), lambda i: (0, 0)),
        ],
        out_specs=[
            pl.BlockSpec((BE, HO), lambda i: (i, 0)),
            pl.BlockSpec((4, BE), lambda i: (0, i)),
        ],
        out_shape=[
            jax.ShapeDtypeStruct((nE, HO), jnp.float32),
            jax.ShapeDtypeStruct((4, nE), jnp.float32),
        ],
    )(f, eT, dT, dnT, m, selT, wp1, bp1, wp2)


def _k8_body(x_ref, pos_ref, agge0_ref, agge1_ref, agga0_ref, agga1_ref,
             ce0_ref, ce1_ref, ca0_ref, ca1_ref,
             w1x_ref, w1a_ref, b1_ref, w2_ref, b2_ref,
             wse_ref, wsa_ref, wsx_ref, bsum_ref,
             gm_ref, gsp_ref, bsp_ref, prelu_ref,
             wpos_ref, bpos_ref,
             ox_ref, op_ref):
    x = x_ref[...]
    xw = jnp.dot(x, w1x_ref[...], preferred_element_type=jnp.float32) + b1_ref[...]

    def upd(agg):
        z = _silu(xw + jnp.dot(agg, w1a_ref[...], preferred_element_type=jnp.float32))
        return jnp.dot(z, w2_ref[...], preferred_element_type=jnp.float32) + b2_ref[...]

    ne = upd(agge0_ref[...] + agge1_ref[...])
    na = upd(agga0_ref[...] + agga1_ref[...])
    h = (jnp.dot(ne, wse_ref[...], preferred_element_type=jnp.float32)
         + jnp.dot(na, wsa_ref[...], preferred_element_type=jnp.float32)
         + jnp.dot(x, wsx_ref[...], preferred_element_type=jnp.float32)
         + bsum_ref[...] + x)
    mu = jnp.dot(h, gm_ref[...], preferred_element_type=jnp.float32)
    d = h - mu
    var = jnp.dot(d * d, gm_ref[...], preferred_element_type=jnp.float32)
    hn = d * jax.lax.rsqrt(var + 1e-5) * gsp_ref[...] + bsp_ref[...]
    ox_ref[...] = jnp.where(hn >= 0, hn, prelu_ref[0, 0] * hn)

    pos = pos_ref[...]
    ce = ce0_ref[...] + ce1_ref[...]
    ca = ca0_ref[...] + ca1_ref[...]
    pe = pos + ce / jnp.maximum(ce[:, 3:4], 1.0)
    pa = pos + ca / jnp.maximum(ca[:, 3:4], 1.0)
    pcat = jnp.concatenate([pe, pa, pos], axis=1)
    op_ref[...] = jnp.dot(pcat, wpos_ref[...], preferred_element_type=jnp.float32) + bpos_ref[...]


def _final(x, pos4, agg_e0, agg_e1, agg_a0, agg_a1, ce0, ce1, ca0, ca1,
           w1x, w1a, b1, w2, b2, wse, wsa, wsx, bsum,
           gm, gsp, bsp, prelu, wpos, bpos):
    grid = (N // BN,)
    full = lambda r, c: pl.BlockSpec((r, c), lambda i: (0, 0))
    nb128 = pl.BlockSpec((BN, HO), lambda i: (i, 0))
    nb4 = pl.BlockSpec((BN, 4), lambda i: (i, 0))
    return pl.pallas_call(
        _k8_body,
        grid=grid,
        in_specs=[
            pl.BlockSpec((BN, D_IN), lambda i: (i, 0)),
            pl.BlockSpec((BN, 4), lambda i: (i, 0)),
            nb128, nb128, nb128, nb128,
            nb4, nb4, nb4, nb4,
            full(D_IN, HO), full(HO, HO), full(1, HO), full(HO, HO), full(1, HO),
            full(HO, HO), full(HO, HO), full(D_IN, HO), full(1, HO),
            full(HO, HO), full(1, HO), full(1, HO),
            pl.BlockSpec(memory_space=pltpu.SMEM),
            full(12, 4), full(1, 4),
        ],
        out_specs=[
            pl.BlockSpec((BN, HO), lambda i: (i, 0)),
            pl.BlockSpec((BN, 4), lambda i: (i, 0)),
        ],
        out_shape=[
            jax.ShapeDtypeStruct((N, HO), jnp.float32),
            jax.ShapeDtypeStruct((N, 4), jnp.float32),
        ],
    )(x, pos4, agg_e0, agg_e1, agg_a0, agg_a1, ce0, ce1, ca0, ca1,
      w1x, w1a, b1, w2, b2, wse, wsa, wsx, bsum,
      gm, gsp, bsp, prelu, wpos, bpos)


def kernel(x, pos, edge_attr, angle_attr, angle_edge_attr, params, edge_index, angle_index):
    p = params
    We1, be1 = p["edge1"]["w"], p["edge1"]["b"]
    Wa1, ba1 = p["ang1"]["w"], p["ang1"]["b"]

    wcat = jnp.concatenate([We1[0:128], We1[128:256], Wa1[0:128], Wa1[128:256], Wa1[256:384]], axis=1)
    bcat = jnp.concatenate([jnp.zeros_like(be1), be1, jnp.zeros_like(ba1), jnp.zeros_like(ba1), ba1])[None]
    tab_ei, tab_ej, tab_ai, tab_ah, tab_aj = _node_tables(x, wcat, bcat)

    pos4 = jnp.pad(pos, ((0, 0), (0, 1)))
    px, py, pz = pos[:, 0], pos[:, 1], pos[:, 2]
    je = edge_index[1]
    ja = angle_index[2]

    sel = jnp.asarray(np.kron(np.eye(H, dtype=np.float32), np.ones((D_OUT, 1), np.float32)))
    selT = sel.T
    zn = jnp.zeros((N,), jnp.float32)
    zN128 = jnp.zeros((N, HO), jnp.float32)

    def attention_path(n_tabs, tabs, idx, jidx, nE, ea, wc, wrad, w2, b2, avec, wp):
        gx, pdT = _make_gather(n_tabs, nE)(*tabs, idx, px, py, pz)
        f, eT, dnT, m = _edge_pass1(gx, pdT, ea, wc, wrad, w2, b2, avec, sel, nE)
        d0, d1, d2, d3 = _make_denom(nE)(eT, jidx, m, zn)
        dT = _make_dgather(nE)(d0[0], d1[0], d2[0], d3[0],
                               d0[1], d1[1], d2[1], d3[1], jidx)
        proj, trT = _edge_pass2(f, eT, dT, dnT, m, selT,
                                wp[0], wp[1], wp[2], nE)
        aggp, c0, c1, c2, c3 = _make_scatter(nE)(proj, trT, jidx, zN128, zn)
        coord0 = jnp.stack([c0[0], c1[0], c2[0], c3[0]], axis=1)
        coord1 = jnp.stack([c0[1], c1[1], c2[1], c3[1]], axis=1)
        return aggp[0:N], aggp[N:2 * N], coord0, coord1

    wp = (p["upd_pos1"]["w"], p["upd_pos1"]["b"][None], p["upd_pos2"]["w"])

    agg_e0, agg_e1, ce0, ce1 = attention_path(
        2, (tab_ei, tab_ej), edge_index, je, E,
        edge_attr, We1[256:272], We1[272:273],
        p["edge2"]["w"], p["edge2"]["b"][None], p["a_edge"].reshape(1, HO), wp)

    ea_a = jnp.concatenate([angle_edge_attr[:, 0], angle_edge_attr[:, 1],
                            angle_attr * (np.pi / 180.0)], axis=1)
    wc_a = jnp.concatenate([Wa1[384:400], Wa1[400:416], Wa1[416:417]], axis=0)
    agg_a0, agg_a1, ca0, ca1 = attention_path(
        3, (tab_ai, tab_ah, tab_aj), angle_index, ja, A,
        ea_a, wc_a, Wa1[417:418],
        p["ang2"]["w"], p["ang2"]["b"][None], p["a_angle"].reshape(1, HO), wp)

    Wu1 = p["upd_node1"]["w"]
    bsum = (p["sum_edge"]["b"] + p["sum_angle"]["b"] + p["sum_x"]["b"]
            + jnp.tile(p["bias"], H))[None]
    gm = jnp.asarray(np.kron(np.eye(H, dtype=np.float32), np.full((D_OUT, D_OUT), 1.0 / D_OUT, np.float32)))
    gsp = jnp.tile(p["ln_g"], H)[None]
    bsp = jnp.tile(p["ln_b"], H)[None]
    prelu = p["prelu"].reshape(1, 1)
    wpos = jnp.zeros((12, 4), jnp.float32)
    wpos = wpos.at[0:3, 0:3].set(p["sum_edge_pos"]["w"])
    wpos = wpos.at[4:7, 0:3].set(p["sum_angle_pos"]["w"])
    wpos = wpos.at[8:11, 0:3].set(p["sum_x_pos"]["w"])
    bpos = jnp.pad(p["sum_edge_pos"]["b"] + p["sum_angle_pos"]["b"] + p["sum_x_pos"]["b"], (0, 1))[None]

    out_x, out_pos4 = _final(
        x, pos4, agg_e0, agg_e1, agg_a0, agg_a1, ce0, ce1, ca0, ca1,
        Wu1[0:128], Wu1[128:256], p["upd_node1"]["b"][None],
        p["upd_node2"]["w"], p["upd_node2"]["b"][None],
        p["sum_edge"]["w"], p["sum_angle"]["w"], p["sum_x"]["w"], bsum,
        gm, gsp, bsp, prelu, wpos, bpos)
    return out_x, out_pos4[:, 0:3]
